# Initial kernel scaffold; baseline (speedup 1.0000x reference)
import jax
import jax.numpy as jnp
from jax.experimental import pallas as pl


def kernel(x, hidden_states, edge_index, Wx_self, Wx_neigh, bx, Wh_self, Wh_neigh, bh, out_W, out_b):
    raise NotImplementedError("write your pallas kernel here")


# speedup vs baseline: 1.0002x; 1.0002x over previous
import jax
import jax.numpy as jnp
from jax.experimental import pallas as pl

N = 10000
S = 6
L = 2
D = 128
H = 128
O = 128

_BLK = 1000


def _proj_body(h_ref, w_ref, b_ref, o_ref):
    o_ref[...] = jnp.dot(h_ref[...], w_ref[...],
                         preferred_element_type=jnp.float32) + b_ref[...]


def _proj(h, w, b):
    return pl.pallas_call(
        _proj_body,
        grid=(N // _BLK,),
        in_specs=[
            pl.BlockSpec((_BLK, H), lambda i: (i, 0)),
            pl.BlockSpec((H, O), lambda i: (0, 0)),
            pl.BlockSpec((1, O), lambda i: (0, 0)),
        ],
        out_specs=pl.BlockSpec((_BLK, O), lambda i: (i, 0)),
        out_shape=jax.ShapeDtypeStruct((N, O), jnp.float32),
    )(h, w, b.reshape(1, O))


def kernel(x, hidden_states, edge_index, Wx_self, Wx_neigh, bx, Wh_self, Wh_neigh, bh, out_W, out_b):
    src = edge_index[0]
    dst = edge_index[1]

    def seg(feat):
        return jax.ops.segment_sum(feat[src], dst, num_segments=N)

    def net(feat, agg, Ws, Wn, b):
        if agg is None:
            agg = seg(feat)
        return feat @ Ws + agg @ Wn + b

    def cell(l, xi, h, x_agg):
        h_agg = seg(h)
        if x_agg is None:
            x_agg = seg(xi)
        r = jax.nn.sigmoid(net(xi, x_agg, Wx_self[l, 0], Wx_neigh[l, 0], bx[l, 0]) + net(h, h_agg, Wh_self[l, 0], Wh_neigh[l, 0], bh[l, 0]))
        u = jax.nn.sigmoid(net(xi, x_agg, Wx_self[l, 1], Wx_neigh[l, 1], bx[l, 1]) + net(h, h_agg, Wh_self[l, 1], Wh_neigh[l, 1], bh[l, 1]))
        h_ = r * h
        c = jnp.tanh(net(xi, x_agg, Wx_self[l, 2], Wx_neigh[l, 2], bx[l, 2]) + net(h_, None, Wh_self[l, 2], Wh_neigh[l, 2], bh[l, 2]))
        return u * h + (1.0 - u) * c

    x_cat = jnp.transpose(x, (1, 0, 2)).reshape(N, S * D)
    x_agg_cat = seg(x_cat)
    x_aggs = [x_agg_cat[:, i * D:(i + 1) * D] for i in range(S)]

    hs = [hidden_states[j] for j in range(L)]
    outputs = []
    for i in range(S):
        inp = x[i]
        x_agg = x_aggs[i]
        new_hs = []
        for j in range(L):
            inp = cell(j, inp, hs[j], x_agg)
            new_hs.append(inp)
            x_agg = None
        outputs.append(_proj(inp, out_W, out_b))
        hs = new_hs
    return jnp.stack(outputs), jnp.stack(hs)


# speedup vs baseline: 1.5983x; 1.5979x over previous
import jax
import jax.numpy as jnp
from jax import lax
from jax.experimental import pallas as pl
from jax.experimental.pallas import tpu as pltpu
from jax.experimental.pallas import tpu_sc as plsc

N = 10000
E = 320000
S = 6
L = 2
D = 128
H = 128
O = 128

_NC, _NS = 2, 16
_NW = _NC * _NS
_CH = 128
_KCH = 80
_EPAD = _NW * _KCH * _CH
_NACC = 10112
_ZROWS = _NACC // _NS


def _seg_body(feat_hbm, src_hbm, dst_hbm, zeros_hbm, out_hbm,
              sidx, didx, buf, acc, sem):
    c = lax.axis_index("c")
    s = lax.axis_index("s")
    wid = s * _NC + c
    pltpu.sync_copy(zeros_hbm.at[pl.ds(0, _ZROWS)],
                    acc.at[pl.ds(s * _ZROWS, _ZROWS)])
    pltpu.sync_copy(src_hbm.at[pl.ds(wid * _KCH, _KCH)], sidx)
    pltpu.sync_copy(dst_hbm.at[pl.ds(wid * _KCH, _KCH)], didx)
    plsc.subcore_barrier()

    def body(g, carry):
        pltpu.async_copy(feat_hbm.at[sidx.at[g]], buf, sem).wait()
        pltpu.sync_copy(buf, acc.at[didx.at[g]], add=True)
        return carry

    lax.fori_loop(0, _KCH, body, 0)
    plsc.subcore_barrier()
    pltpu.sync_copy(acc.at[pl.ds(s * _ZROWS, _ZROWS)],
                    out_hbm.at[c, pl.ds(s * _ZROWS, _ZROWS)])


_seg_call = pl.kernel(
    _seg_body,
    out_type=jax.ShapeDtypeStruct((_NC, _NACC, H), jnp.float32),
    mesh=plsc.VectorSubcoreMesh(core_axis_name="c", subcore_axis_name="s"),
    scratch_types=[
        pltpu.VMEM((_KCH, _CH), jnp.int32),
        pltpu.VMEM((_KCH, _CH), jnp.int32),
        pltpu.VMEM((_CH, H), jnp.float32),
        pltpu.VMEM_SHARED((_NACC, H), jnp.float32),
        pltpu.SemaphoreType.DMA,
    ],
)

_BLK = 1000


def _proj_body(h_ref, w_ref, b_ref, o_ref):
    o_ref[...] = jnp.dot(h_ref[...], w_ref[...],
                         preferred_element_type=jnp.float32) + b_ref[...]


def _proj(h, w, b):
    return pl.pallas_call(
        _proj_body,
        grid=(N // _BLK,),
        in_specs=[
            pl.BlockSpec((_BLK, H), lambda i: (i, 0)),
            pl.BlockSpec((H, O), lambda i: (0, 0)),
            pl.BlockSpec((1, O), lambda i: (0, 0)),
        ],
        out_specs=pl.BlockSpec((_BLK, O), lambda i: (i, 0)),
        out_shape=jax.ShapeDtypeStruct((N, O), jnp.float32),
    )(h, w, b.reshape(1, O))


def kernel(x, hidden_states, edge_index, Wx_self, Wx_neigh, bx, Wh_self, Wh_neigh, bh, out_W, out_b):
    src = edge_index[0]
    dst = edge_index[1]
    pad = _EPAD - E
    src_p = jnp.concatenate([src, jnp.zeros((pad,), jnp.int32)]).reshape(_NW * _KCH, _CH)
    dst_p = jnp.concatenate([dst, jnp.full((pad,), N, jnp.int32)]).reshape(_NW * _KCH, _CH)
    zeros = jnp.zeros((_ZROWS, H), jnp.float32)

    def seg(feat):
        parts = _seg_call(feat, src_p, dst_p, zeros)
        return parts[0, :N] + parts[1, :N]

    def net(feat, agg, Ws, Wn, b):
        if agg is None:
            agg = seg(feat)
        return feat @ Ws + agg @ Wn + b

    def cell(l, xi, h, x_agg):
        h_agg = seg(h)
        if x_agg is None:
            x_agg = seg(xi)
        r = jax.nn.sigmoid(net(xi, x_agg, Wx_self[l, 0], Wx_neigh[l, 0], bx[l, 0]) + net(h, h_agg, Wh_self[l, 0], Wh_neigh[l, 0], bh[l, 0]))
        u = jax.nn.sigmoid(net(xi, x_agg, Wx_self[l, 1], Wx_neigh[l, 1], bx[l, 1]) + net(h, h_agg, Wh_self[l, 1], Wh_neigh[l, 1], bh[l, 1]))
        h_ = r * h
        c = jnp.tanh(net(xi, x_agg, Wx_self[l, 2], Wx_neigh[l, 2], bx[l, 2]) + net(h_, None, Wh_self[l, 2], Wh_neigh[l, 2], bh[l, 2]))
        return u * h + (1.0 - u) * c

    x_aggs = [seg(x[i]) for i in range(S)]

    hs = [hidden_states[j] for j in range(L)]
    outputs = []
    for i in range(S):
        inp = x[i]
        x_agg = x_aggs[i]
        new_hs = []
        for j in range(L):
            inp = cell(j, inp, hs[j], x_agg)
            new_hs.append(inp)
            x_agg = None
        outputs.append(_proj(inp, out_W, out_b))
        hs = new_hs
    return jnp.stack(outputs), jnp.stack(hs)


# speedup vs baseline: 1.7391x; 1.0881x over previous
import jax
import jax.numpy as jnp
from jax import lax
from jax.experimental import pallas as pl
from jax.experimental.pallas import tpu as pltpu
from jax.experimental.pallas import tpu_sc as plsc

N = 10000
E = 320000
S = 6
L = 2
D = 128
H = 128
O = 128

_NC, _NS = 2, 16
_NW = _NC * _NS
_CH = 128
_KCH = 80
_EPAD = _NW * _KCH * _CH
_NACC = 10112
_ZROWS = _NACC // _NS


_NB = 2
_NHALF = 2
_HKCH = _KCH // _NHALF


def _seg_body(feat_hbm, src_hbm, dst_hbm, zeros_hbm, out_hbm,
              sidx, didx, buf, acc, gs0, gs1, ss0, ss1):
    gsems = (gs0, gs1)
    ssems = (ss0, ss1)
    c = lax.axis_index("c")
    s = lax.axis_index("s")
    wid = s * _NC + c
    pltpu.sync_copy(zeros_hbm.at[pl.ds(0, _ZROWS)],
                    acc.at[pl.ds(s * _ZROWS, _ZROWS)])
    plsc.subcore_barrier()

    def start_gather(g, b):
        pltpu.async_copy(feat_hbm.at[sidx.at[g]], buf.at[b], gsems[b])

    def wait_gather(g, b):
        pltpu.make_async_copy(feat_hbm.at[sidx.at[g]], buf.at[b],
                              gsems[b]).wait()

    for half in range(_NHALF):
        cb = wid * _KCH + half * _HKCH
        pltpu.sync_copy(src_hbm.at[pl.ds(cb, _HKCH)], sidx)
        pltpu.sync_copy(dst_hbm.at[pl.ds(cb, _HKCH)], didx)

        for b in range(_NB):
            start_gather(b, b)

        def gg_body(gg, carry):
            g0 = gg * _NB
            descs = []
            for b in range(_NB):
                wait_gather(g0 + b, b)
                descs.append(pltpu.async_copy(
                    buf.at[b], acc.at[didx.at[g0 + b]], ssems[b], add=True))
            for b in range(_NB):
                descs[b].wait()
                start_gather(g0 + b + _NB, b)
            return carry

        lax.fori_loop(0, _HKCH // _NB - 1, gg_body, 0)

        g0 = _HKCH - _NB
        descs = []
        for b in range(_NB):
            wait_gather(g0 + b, b)
            descs.append(pltpu.async_copy(
                buf.at[b], acc.at[didx.at[g0 + b]], ssems[b], add=True))
        for b in range(_NB):
            descs[b].wait()

    plsc.subcore_barrier()
    pltpu.sync_copy(acc.at[pl.ds(s * _ZROWS, _ZROWS)],
                    out_hbm.at[c, pl.ds(s * _ZROWS, _ZROWS)])


_seg_call = pl.kernel(
    _seg_body,
    out_type=jax.ShapeDtypeStruct((_NC, _NACC, H), jnp.float32),
    mesh=plsc.VectorSubcoreMesh(core_axis_name="c", subcore_axis_name="s"),
    scratch_types=[
        pltpu.VMEM((_HKCH, _CH), jnp.int32),
        pltpu.VMEM((_HKCH, _CH), jnp.int32),
        pltpu.VMEM((_NB, _CH, H), jnp.float32),
        pltpu.VMEM_SHARED((_NACC, H), jnp.float32),
    ] + [pltpu.SemaphoreType.DMA] * (2 * _NB),
)

_BLK = 1000


def _proj_body(h_ref, w_ref, b_ref, o_ref):
    o_ref[...] = jnp.dot(h_ref[...], w_ref[...],
                         preferred_element_type=jnp.float32) + b_ref[...]


def _proj(h, w, b):
    return pl.pallas_call(
        _proj_body,
        grid=(N // _BLK,),
        in_specs=[
            pl.BlockSpec((_BLK, H), lambda i: (i, 0)),
            pl.BlockSpec((H, O), lambda i: (0, 0)),
            pl.BlockSpec((1, O), lambda i: (0, 0)),
        ],
        out_specs=pl.BlockSpec((_BLK, O), lambda i: (i, 0)),
        out_shape=jax.ShapeDtypeStruct((N, O), jnp.float32),
    )(h, w, b.reshape(1, O))


def kernel(x, hidden_states, edge_index, Wx_self, Wx_neigh, bx, Wh_self, Wh_neigh, bh, out_W, out_b):
    src = edge_index[0]
    dst = edge_index[1]
    pad = _EPAD - E
    src_p = jnp.concatenate([src, jnp.zeros((pad,), jnp.int32)]).reshape(_NW * _KCH, _CH)
    dst_p = jnp.concatenate([dst, jnp.full((pad,), N, jnp.int32)]).reshape(_NW * _KCH, _CH)
    zeros = jnp.zeros((_ZROWS, H), jnp.float32)

    def seg(feat):
        parts = _seg_call(feat, src_p, dst_p, zeros)
        return parts[0, :N] + parts[1, :N]

    def net(feat, agg, Ws, Wn, b):
        if agg is None:
            agg = seg(feat)
        return feat @ Ws + agg @ Wn + b

    def cell(l, xi, h, x_agg):
        h_agg = seg(h)
        if x_agg is None:
            x_agg = seg(xi)
        r = jax.nn.sigmoid(net(xi, x_agg, Wx_self[l, 0], Wx_neigh[l, 0], bx[l, 0]) + net(h, h_agg, Wh_self[l, 0], Wh_neigh[l, 0], bh[l, 0]))
        u = jax.nn.sigmoid(net(xi, x_agg, Wx_self[l, 1], Wx_neigh[l, 1], bx[l, 1]) + net(h, h_agg, Wh_self[l, 1], Wh_neigh[l, 1], bh[l, 1]))
        h_ = r * h
        c = jnp.tanh(net(xi, x_agg, Wx_self[l, 2], Wx_neigh[l, 2], bx[l, 2]) + net(h_, None, Wh_self[l, 2], Wh_neigh[l, 2], bh[l, 2]))
        return u * h + (1.0 - u) * c

    x_aggs = [seg(x[i]) for i in range(S)]

    hs = [hidden_states[j] for j in range(L)]
    outputs = []
    for i in range(S):
        inp = x[i]
        x_agg = x_aggs[i]
        new_hs = []
        for j in range(L):
            inp = cell(j, inp, hs[j], x_agg)
            new_hs.append(inp)
            x_agg = None
        outputs.append(_proj(inp, out_W, out_b))
        hs = new_hs
    return jnp.stack(outputs), jnp.stack(hs)
